# baseline (speedup 1.0000x reference)
import jax
import jax.numpy as jnp
from jax import lax
from jax.experimental import pallas as pl
from jax.experimental.pallas import tpu as pltpu
from jax.experimental.pallas import tpu_sc as plsc

BATCH = 16384
DIM = 64
NW = 32
B_PER_W = BATCH // NW
ROWS_PER_W = 2 * B_PER_W
CHUNK = 128
NCHUNK = ROWS_PER_W // CHUNK
NBLK = B_PER_W // 16


def _rsqrt_newton(s):
    i = plsc.bitcast(s, jnp.int32)
    y = plsc.bitcast(jnp.int32(0x5F3759DF) - (i >> 1), jnp.float32)
    half = s * 0.5
    for _ in range(3):
        y = y * (1.5 - half * y * y)
    return jnp.minimum(y, 1e12)


def _body(x_hbm, w_hbm, out_hbm, idx_v, rows_v, sums_v, out_v, sem):
    wid = lax.axis_index("s") * 2 + lax.axis_index("c")

    pltpu.sync_copy(x_hbm.at[wid], idx_v)

    copies = [
        pltpu.async_copy(
            w_hbm.at[idx_v.at[j]], rows_v.at[pl.ds(j * CHUNK, CHUNK)], sem
        )
        for j in range(NCHUNK)
    ]
    for c in copies:
        c.wait()

    lanes = lax.iota(jnp.int32, 16)
    last = lanes == 15

    def e_body(i, _):
        a = [rows_v[2 * i, pl.ds(k * 16, 16)] for k in range(4)]
        b = [rows_v[2 * i + 1, pl.ds(k * 16, 16)] for k in range(4)]
        p = a[0] * b[0] + a[1] * b[1] + a[2] * b[2] + a[3] * b[3]
        q = a[0] * a[0] + a[1] * a[1] + a[2] * a[2] + a[3] * a[3]
        r = b[0] * b[0] + b[1] * b[1] + b[2] * b[2] + b[3] * b[3]
        base = jnp.full((16,), 0, jnp.int32) + i
        plsc.store_scatter(sums_v, [base], plsc.cumsum(p), mask=last)
        plsc.store_scatter(
            sums_v, [base + B_PER_W], plsc.cumsum(q), mask=last)
        plsc.store_scatter(
            sums_v, [base + 2 * B_PER_W], plsc.cumsum(r), mask=last)
        return 0

    lax.fori_loop(0, B_PER_W, e_body, 0, unroll=4)

    def blk_body(blk, _):
        sl = pl.ds(blk * 16, 16)
        s01 = sums_v[sl]
        s00 = sums_v[pl.ds(B_PER_W + blk * 16, 16)]
        s11 = sums_v[pl.ds(2 * B_PER_W + blk * 16, 16)]
        out_v[sl] = s01 * _rsqrt_newton(s00) * _rsqrt_newton(s11)
        return 0

    lax.fori_loop(0, NBLK, blk_body, 0)

    pltpu.sync_copy(out_v, out_hbm.at[pl.ds(wid * B_PER_W, B_PER_W)])


def kernel(x, W):
    x3 = x.astype(jnp.int32).reshape(NW, NCHUNK, CHUNK)
    mesh = plsc.VectorSubcoreMesh(core_axis_name="c", subcore_axis_name="s")
    out = pl.kernel(
        _body,
        mesh=mesh,
        compiler_params=pltpu.CompilerParams(
            needs_layout_passes=False, use_tc_tiling_on_sc=False
        ),
        out_type=jax.ShapeDtypeStruct((BATCH,), jnp.float32),
        scratch_types=[
            pltpu.VMEM((NCHUNK, CHUNK), jnp.int32),
            pltpu.VMEM((ROWS_PER_W, DIM), jnp.float32),
            pltpu.VMEM((3 * B_PER_W,), jnp.float32),
            pltpu.VMEM((B_PER_W,), jnp.float32),
            pltpu.SemaphoreType.DMA,
        ],
    )(x3, W)
    return out[:, None]
